# baseline (speedup 1.0000x reference)
import functools

import jax
import jax.numpy as jnp
from jax import lax
from jax.experimental import pallas as pl
from jax.experimental.pallas import tpu as pltpu
from jax.experimental.pallas import tpu_sc as plsc

L = 16
NS = 16
NC = 2
NW = NC * NS
CHUNK = 4096
ROWS = CHUNK // 128


def _prep_body(b_ref, p_ref, v_ref):
    mask = b_ref[...] > 0.5
    packed = (p_ref[...] << 1) | 1
    v_ref[...] = jnp.where(mask, packed, 0).astype(jnp.int8)


def _apply_body(b_ref, q_ref, a0_ref, a1_ref, c0_ref, c1_ref, o_ref):
    b = b_ref[...]
    q = q_ref[...]
    a = (a0_ref[...] + a1_ref[...]).astype(jnp.float32)
    cnt = c0_ref[...] + c1_ref[...]
    t = jnp.exp(a * jnp.log((1.0 - q) / q))
    den = b + (1.0 - b) * t
    post = jnp.where(den > 0.0, b / den, b)
    o_ref[...] = jnp.where(cnt > 0, post, b)


def _make_sc_kernel(n, n_pad, e):
    i32 = jnp.int32
    sl = n_pad // NS
    w_edges = e // NW
    full = w_edges // CHUNK
    tail = w_edges % CHUNK
    tgroups = tail // L
    trows = -(-tail // 128) if tail else 0
    mesh = plsc.VectorSubcoreMesh(
        core_axis_name="c", subcore_axis_name="s",
        num_cores=NC, num_subcores=NS)

    @functools.partial(
        pl.kernel,
        out_type=(jax.ShapeDtypeStruct((NC, n_pad), jnp.int32),
                  jax.ShapeDtypeStruct((NC, n_pad), jnp.int32)),
        mesh=mesh,
        compiler_params=pltpu.CompilerParams(needs_layout_passes=False),
        scratch_types=[
            pltpu.VMEM((n_pad // 4,), jnp.int32),
            [pltpu.VMEM((2 * CHUNK,), jnp.int32) for _ in range(2)],
            [pltpu.VMEM((2 * CHUNK,), jnp.int32) for _ in range(2)],
            [pltpu.VMEM((CHUNK,), jnp.int32) for _ in range(2)],
            [pltpu.VMEM((CHUNK,), jnp.int32) for _ in range(2)],
            [pltpu.VMEM((CHUNK,), jnp.int32) for _ in range(2)],
            [pltpu.SemaphoreType.DMA for _ in range(2)],
            [pltpu.SemaphoreType.DMA for _ in range(2)],
            pltpu.VMEM_SHARED((n_pad,), jnp.int32),
            pltpu.VMEM_SHARED((n_pad,), jnp.int32),
        ],
    )
    def sc_kernel(vpk_hbm, e32_hbm, zer_hbm, aout, cout,
                  table, s64, d64, dix, aval, cval, sem_stage, sem_scat,
                  acc_a, acc_c):
        c = lax.axis_index("c")
        s = lax.axis_index("s")
        w = c * i32(NS) + s
        wbase2 = w * i32(2 * w_edges)
        iota2 = lax.iota(jnp.int32, L) * i32(2)

        pltpu.sync_copy(zer_hbm.at[pl.ds(s * i32(sl), sl)],
                        acc_a.at[pl.ds(s * i32(sl), sl)])
        pltpu.sync_copy(zer_hbm.at[pl.ds(s * i32(sl), sl)],
                        acc_c.at[pl.ds(s * i32(sl), sl)])
        pltpu.sync_copy(vpk_hbm, table)
        plsc.subcore_barrier()

        def fire_stage(k, p, nwords):
            off = wbase2 + i32(2 * k * CHUNK)
            return (
                pltpu.async_copy(e32_hbm.at[pl.ds(off, nwords)],
                                 s64[p].at[pl.ds(0, nwords)], sem_stage[p]),
                pltpu.async_copy(e32_hbm.at[pl.ds(off + i32(2 * e), nwords)],
                                 d64[p].at[pl.ds(0, nwords)], sem_stage[p]),
            )

        def build_group(p, r, g):
            ii = iota2 + (r * i32(256) + i32(g * 32))
            sidx = plsc.load_gather(s64[p], [ii])
            didx = plsc.load_gather(d64[p], [ii])
            word = plsc.load_gather(table, [sidx >> i32(2)])
            v8 = (word >> ((sidx & i32(3)) << i32(3))) & i32(0xFF)
            cnt = v8 & i32(1)
            a = i32(2) * (v8 >> i32(1)) - i32(10) * cnt
            o = r * i32(128) + i32(g * 16)
            dix[p][pl.ds(o, 16)] = didx
            aval[p][pl.ds(o, 16)] = a
            cval[p][pl.ds(o, 16)] = cnt

        def build_rows(p, nrows):
            def row_body(r, carry):
                for g in range(8):
                    build_group(p, r, g)
                return carry
            lax.fori_loop(i32(0), i32(nrows), row_body, i32(0))

        def fire_scatter(p):
            return (
                pltpu.async_copy(aval[p], acc_a.at[dix[p]],
                                 sem_scat[p], add=True),
                pltpu.async_copy(cval[p], acc_c.at[dix[p]],
                                 sem_scat[p], add=True),
            )

        stage_d = [None, None]
        scat_d = [None, None]
        if full > 0:
            stage_d[0] = fire_stage(0, 0, 2 * CHUNK)
        if full > 1:
            stage_d[1] = fire_stage(1, 1, 2 * CHUNK)

        for k in range(full):
            p = k % 2
            if scat_d[p] is not None:
                scat_d[p][0].wait()
                scat_d[p][1].wait()
                scat_d[p] = None
            stage_d[p][0].wait()
            stage_d[p][1].wait()
            stage_d[p] = None
            build_rows(p, ROWS)
            if k + 2 < full:
                stage_d[p] = fire_stage(k + 2, p, 2 * CHUNK)
            elif tail and k + 2 == full:
                stage_d[p] = fire_stage(full, p, 2 * tail)
            scat_d[p] = fire_scatter(p)

        if tail:
            p = full % 2
            if stage_d[p] is None:
                stage_d[p] = fire_stage(full, p, 2 * tail)
            if scat_d[p] is not None:
                scat_d[p][0].wait()
                scat_d[p][1].wait()
                scat_d[p] = None
            stage_d[p][0].wait()
            stage_d[p][1].wait()
            if tgroups // 8 > 0:
                build_rows(p, tgroups // 8)
            for g8 in range(8 * (tgroups // 8), tgroups):
                build_group(p, i32(g8 // 8), g8 % 8)
            zero16 = jnp.zeros((L,), jnp.int32)
            dummy16 = jnp.full((L,), n, jnp.int32)

            def fill_body(g16, carry):
                o = g16 * i32(16)
                dix[p][pl.ds(o, 16)] = dummy16
                aval[p][pl.ds(o, 16)] = zero16
                cval[p][pl.ds(o, 16)] = zero16
                return carry
            lax.fori_loop(i32(tgroups), i32(CHUNK // L), fill_body, i32(0))
            scat_d[p] = fire_scatter(p)

        for p in range(2):
            if scat_d[p] is not None:
                scat_d[p][0].wait()
                scat_d[p][1].wait()

        plsc.subcore_barrier()
        pltpu.sync_copy(acc_a.at[pl.ds(s * i32(sl), sl)],
                        aout.at[c, pl.ds(s * i32(sl), sl)])
        pltpu.sync_copy(acc_c.at[pl.ds(s * i32(sl), sl)],
                        cout.at[c, pl.ds(s * i32(sl), sl)])

    return sc_kernel


def kernel(belief, probability, payoff_sample, edge_index):
    n = belief.shape[0]
    e = edge_index.shape[1]
    assert e % (NW * L) == 0, "edge count must divide into 16-edge groups"
    n_pad = -(-n // (NS * 128)) * (NS * 128)
    rows2d = n_pad // 128

    b32 = belief.astype(jnp.float32)
    q32 = probability.astype(jnp.float32)
    p32 = payoff_sample.astype(jnp.int32)

    b_pad = jnp.pad(b32, (0, n_pad - n)).reshape(rows2d, 128)
    q_pad = jnp.pad(q32, (0, n_pad - n), constant_values=0.5).reshape(rows2d, 128)
    p_pad = jnp.pad(p32, (0, n_pad - n)).reshape(rows2d, 128)
    e32 = lax.bitcast_convert_type(edge_index, jnp.int32).reshape(4 * e)
    zer = jnp.zeros((n_pad,), jnp.int32)

    vpk = pl.pallas_call(
        _prep_body,
        out_shape=jax.ShapeDtypeStruct((rows2d, 128), jnp.int8),
    )(b_pad, p_pad)
    tbl32 = lax.bitcast_convert_type(
        vpk.reshape(n_pad // 4, 4), jnp.int32).reshape(n_pad // 4)

    acc_a, acc_c = _make_sc_kernel(n, n_pad, e)(tbl32, e32, zer)

    out = pl.pallas_call(
        _apply_body,
        out_shape=jax.ShapeDtypeStruct((rows2d, 128), jnp.float32),
    )(b_pad, q_pad,
      acc_a[0].reshape(rows2d, 128), acc_a[1].reshape(rows2d, 128),
      acc_c[0].reshape(rows2d, 128), acc_c[1].reshape(rows2d, 128))

    return out.reshape(n_pad)[:n].astype(jnp.float64)


# speedup vs baseline: 13.1528x; 13.1528x over previous
import functools

import jax
import jax.numpy as jnp
from jax import lax
from jax.experimental import pallas as pl
from jax.experimental.pallas import tpu as pltpu
from jax.experimental.pallas import tpu_sc as plsc

L = 16
NS = 16
NC = 2
NW = NC * NS
RB = 32
CHUNK = RB * 128


def _prep_body(b_ref, p_ref, v_ref):
    mask = b_ref[...] > 0.5
    packed = (p_ref[...] << 1) | 1
    v_ref[...] = jnp.where(mask, packed, 0).astype(jnp.int8)


def _apply_body(b_ref, q_ref, a0_ref, a1_ref, c0_ref, c1_ref, o_ref):
    b = b_ref[...]
    q = q_ref[...]
    a = (a0_ref[...] + a1_ref[...]).astype(jnp.float32)
    cnt = c0_ref[...] + c1_ref[...]
    t = jnp.exp(a * jnp.log((1.0 - q) / q))
    den = b + (1.0 - b) * t
    post = jnp.where(den > 0.0, b / den, b)
    o_ref[...] = jnp.where(cnt > 0, post, b)


def _make_sc_kernel(n, n_pad, e):
    i32 = jnp.int32
    sl = n_pad // NS
    rows_total = e // 128
    rpw = rows_total // NW
    rem = rows_total % NW
    full = rpw // RB
    tail = rpw % RB
    mesh = plsc.VectorSubcoreMesh(
        core_axis_name="c", subcore_axis_name="s",
        num_cores=NC, num_subcores=NS)

    @functools.partial(
        pl.kernel,
        out_type=(jax.ShapeDtypeStruct((NC, n_pad), jnp.int32),
                  jax.ShapeDtypeStruct((NC, n_pad), jnp.int32)),
        mesh=mesh,
        compiler_params=pltpu.CompilerParams(needs_layout_passes=False),
        scratch_types=[
            pltpu.VMEM((n_pad // 4,), jnp.int32),
            [pltpu.VMEM((CHUNK,), jnp.int32) for _ in range(2)],
            [pltpu.VMEM((CHUNK,), jnp.int32) for _ in range(2)],
            [pltpu.VMEM((CHUNK,), jnp.int32) for _ in range(2)],
            [pltpu.VMEM((CHUNK,), jnp.int32) for _ in range(2)],
            [pltpu.SemaphoreType.DMA for _ in range(2)],
            pltpu.VMEM_SHARED((n_pad,), jnp.int32),
            pltpu.VMEM_SHARED((n_pad,), jnp.int32),
        ],
    )
    def sc_kernel(vpk_hbm, src_hbm, dst_hbm, zer_hbm, aout, cout,
                  table, six, dix, aval, cval, sem_stage, acc_a, acc_c):
        c = lax.axis_index("c")
        s = lax.axis_index("s")
        w = c * i32(NS) + s
        row0 = w * i32(rpw) + jnp.minimum(w, i32(rem))
        has_extra = w < i32(rem)

        pltpu.sync_copy(zer_hbm.at[pl.ds(s * i32(sl), sl)],
                        acc_a.at[pl.ds(s * i32(sl), sl)])
        pltpu.sync_copy(zer_hbm.at[pl.ds(s * i32(sl), sl)],
                        acc_c.at[pl.ds(s * i32(sl), sl)])
        pltpu.sync_copy(vpk_hbm, table)
        plsc.subcore_barrier()

        def fire_stage(k, p, nrows):
            r = row0 + i32(k * RB)
            return (
                pltpu.async_copy(src_hbm.at[pl.ds(r * i32(128), nrows * 128)],
                                 six[p].at[pl.ds(0, nrows * 128)],
                                 sem_stage[p]),
                pltpu.async_copy(dst_hbm.at[pl.ds(r * i32(128), nrows * 128)],
                                 dix[p].at[pl.ds(0, nrows * 128)],
                                 sem_stage[p]),
            )

        def build_group(p, r, g):
            sidx = six[p][pl.ds(r * i32(128) + i32(g * 16), 16)]
            word = plsc.load_gather(table, [sidx >> i32(2)])
            v8 = (word >> ((sidx & i32(3)) << i32(3))) & i32(0xFF)
            cnt = v8 & i32(1)
            a = i32(2) * (v8 >> i32(1)) - i32(10) * cnt
            o = r * i32(128) + i32(g * 16)
            aval[p][pl.ds(o, 16)] = a
            cval[p][pl.ds(o, 16)] = cnt

        def build_rows(p, nrows):
            def row_body(r, carry):
                for g in range(8):
                    build_group(p, r, g)
                return carry
            lax.fori_loop(i32(0), i32(nrows), row_body, i32(0))

        def fire_scatter(p, nrows):
            for j in range(nrows):
                pltpu.sync_copy(aval[p].at[pl.ds(j * 128, 128)],
                                acc_a.at[dix[p].at[pl.ds(j * 128, 128)]],
                                add=True)
                pltpu.sync_copy(cval[p].at[pl.ds(j * 128, 128)],
                                acc_c.at[dix[p].at[pl.ds(j * 128, 128)]],
                                add=True)

        stage_d = [None, None]
        if full > 0:
            stage_d[0] = fire_stage(0, 0, RB)
        if full > 1:
            stage_d[1] = fire_stage(1, 1, RB)

        for k in range(full):
            p = k % 2
            stage_d[p][0].wait()
            stage_d[p][1].wait()
            stage_d[p] = None
            build_rows(p, RB)
            if k + 2 < full:
                stage_d[p] = fire_stage(k + 2, p, RB)
            elif tail and k + 2 == full:
                stage_d[p] = fire_stage(full, p, tail)
            fire_scatter(p, RB)

        if tail:
            p = full % 2
            if stage_d[p] is None:
                stage_d[p] = fire_stage(full, p, tail)
            stage_d[p][0].wait()
            stage_d[p][1].wait()
            build_rows(p, tail)
            fire_scatter(p, tail)

        if rem:
            @pl.when(has_extra)
            def _extra():
                r = row0 + i32(rpw)
                pltpu.sync_copy(src_hbm.at[pl.ds(r * i32(128), 128)],
                                six[0].at[pl.ds(0, 128)])
                pltpu.sync_copy(dst_hbm.at[pl.ds(r * i32(128), 128)],
                                dix[0].at[pl.ds(0, 128)])
                for g in range(8):
                    build_group(0, i32(0), g)
                fire_scatter(0, 1)

        plsc.subcore_barrier()
        pltpu.sync_copy(acc_a.at[pl.ds(s * i32(sl), sl)],
                        aout.at[c, pl.ds(s * i32(sl), sl)])
        pltpu.sync_copy(acc_c.at[pl.ds(s * i32(sl), sl)],
                        cout.at[c, pl.ds(s * i32(sl), sl)])

    return sc_kernel


def kernel(belief, probability, payoff_sample, edge_index):
    n = belief.shape[0]
    e = edge_index.shape[1]
    assert e % 128 == 0, "edge count must be a multiple of 128"
    n_pad = -(-n // (NS * 128)) * (NS * 128)
    rows2d = n_pad // 128

    b32 = belief.astype(jnp.float32)
    q32 = probability.astype(jnp.float32)
    p32 = payoff_sample.astype(jnp.int32)

    b_pad = jnp.pad(b32, (0, n_pad - n)).reshape(rows2d, 128)
    q_pad = jnp.pad(q32, (0, n_pad - n), constant_values=0.5).reshape(rows2d, 128)
    p_pad = jnp.pad(p32, (0, n_pad - n)).reshape(rows2d, 128)
    src32 = edge_index[0].astype(jnp.int32)
    dst32 = edge_index[1].astype(jnp.int32)
    zer = jnp.zeros((n_pad,), jnp.int32)

    vpk = pl.pallas_call(
        _prep_body,
        out_shape=jax.ShapeDtypeStruct((rows2d, 128), jnp.int8),
    )(b_pad, p_pad)
    tbl32 = lax.bitcast_convert_type(
        vpk.reshape(n_pad // 4, 4), jnp.int32).reshape(n_pad // 4)

    acc_a, acc_c = _make_sc_kernel(n, n_pad, e)(tbl32, src32, dst32, zer)

    out = pl.pallas_call(
        _apply_body,
        out_shape=jax.ShapeDtypeStruct((rows2d, 128), jnp.float32),
    )(b_pad, q_pad,
      acc_a[0].reshape(rows2d, 128), acc_a[1].reshape(rows2d, 128),
      acc_c[0].reshape(rows2d, 128), acc_c[1].reshape(rows2d, 128))

    return out.reshape(n_pad)[:n].astype(jnp.float64)


# speedup vs baseline: 15.6513x; 1.1900x over previous
import functools

import jax
import jax.numpy as jnp
from jax import lax
from jax.experimental import pallas as pl
from jax.experimental.pallas import tpu as pltpu
from jax.experimental.pallas import tpu_sc as plsc

L = 16
NS = 16
NC = 2
NW = NC * NS
RB = 32
CHUNK = RB * 128
SCW = 512


def _prep_body(b0, b1, b2, b3, p0, p1, p2, p3, v_ref):
    def byte(b_ref, p_ref):
        mask = b_ref[...] > 0.5
        return jnp.where(mask, (p_ref[...] << 1) | 1, 0)
    v_ref[...] = (byte(b0, p0) | (byte(b1, p1) << 8)
                  | (byte(b2, p2) << 16) | (byte(b3, p3) << 24))


def _apply_body(b_ref, q_ref, a0_ref, a1_ref, c0_ref, c1_ref, o_ref):
    b = b_ref[...]
    q = q_ref[...]
    a = (a0_ref[...] + a1_ref[...]).astype(jnp.float32)
    cnt = c0_ref[...] + c1_ref[...]
    t = jnp.exp(a * jnp.log((1.0 - q) / q))
    den = b + (1.0 - b) * t
    post = jnp.where(den > 0.0, b / den, b)
    o_ref[...] = jnp.where(cnt > 0, post, b)


def _make_sc_kernel(n, n_pad, e):
    i32 = jnp.int32
    N4LOG = (n_pad // 4).bit_length() - 1
    sl = n_pad // NS
    rows_total = e // 128
    rpw = rows_total // NW
    rem = rows_total % NW
    full = rpw // RB
    tail = rpw % RB
    mesh = plsc.VectorSubcoreMesh(
        core_axis_name="c", subcore_axis_name="s",
        num_cores=NC, num_subcores=NS)

    @functools.partial(
        pl.kernel,
        out_type=(jax.ShapeDtypeStruct((NC, n_pad), jnp.int32),
                  jax.ShapeDtypeStruct((NC, n_pad), jnp.int32)),
        mesh=mesh,
        compiler_params=pltpu.CompilerParams(needs_layout_passes=False),
        scratch_types=[
            pltpu.VMEM((n_pad // 4,), jnp.int32),
            [pltpu.VMEM((CHUNK,), jnp.int32) for _ in range(2)],
            [pltpu.VMEM((CHUNK,), jnp.int32) for _ in range(2)],
            [pltpu.VMEM((CHUNK,), jnp.int32) for _ in range(2)],
            [pltpu.VMEM((CHUNK,), jnp.int32) for _ in range(2)],
            [pltpu.SemaphoreType.DMA for _ in range(2)],
            pltpu.VMEM_SHARED((n_pad,), jnp.int32),
            pltpu.VMEM_SHARED((n_pad,), jnp.int32),
        ],
    )
    def sc_kernel(vpk_hbm, src_hbm, dst_hbm, zer_hbm, aout, cout,
                  table, six, dix, aval, cval, sem_stage, acc_a, acc_c):
        c = lax.axis_index("c")
        s = lax.axis_index("s")
        w = c * i32(NS) + s
        row0 = w * i32(rpw) + jnp.minimum(w, i32(rem))
        has_extra = w < i32(rem)

        pltpu.sync_copy(zer_hbm.at[pl.ds(s * i32(sl), sl)],
                        acc_a.at[pl.ds(s * i32(sl), sl)])
        pltpu.sync_copy(zer_hbm.at[pl.ds(s * i32(sl), sl)],
                        acc_c.at[pl.ds(s * i32(sl), sl)])
        pltpu.sync_copy(vpk_hbm, table)
        plsc.subcore_barrier()

        def fire_stage(k, p, nrows):
            r = row0 + i32(k * RB)
            return (
                pltpu.async_copy(src_hbm.at[pl.ds(r * i32(128), nrows * 128)],
                                 six[p].at[pl.ds(0, nrows * 128)],
                                 sem_stage[p]),
                pltpu.async_copy(dst_hbm.at[pl.ds(r * i32(128), nrows * 128)],
                                 dix[p].at[pl.ds(0, nrows * 128)],
                                 sem_stage[p]),
            )

        def build_group(p, r, g):
            sidx = six[p][pl.ds(r * i32(128) + i32(g * 16), 16)]
            word = plsc.load_gather(table, [sidx & i32(n_pad // 4 - 1)])
            v8 = (word >> ((sidx >> i32(N4LOG)) << i32(3))) & i32(0xFF)
            cnt = v8 & i32(1)
            a = i32(2) * (v8 >> i32(1)) - i32(10) * cnt
            o = r * i32(128) + i32(g * 16)
            aval[p][pl.ds(o, 16)] = a
            cval[p][pl.ds(o, 16)] = cnt

        def build_rows(p, nrows):
            def row_body(r, carry):
                for g in range(8):
                    build_group(p, r, g)
                return carry
            lax.fori_loop(i32(0), i32(nrows), row_body, i32(0))

        def fire_scatter(p, nrows):
            nedges = nrows * 128
            nsc = -(-nedges // SCW)
            for j in range(nsc):
                wdt = min(SCW, nedges - j * SCW)
                pltpu.sync_copy(aval[p].at[pl.ds(j * SCW, wdt)],
                                acc_a.at[dix[p].at[pl.ds(j * SCW, wdt)]],
                                add=True)
                pltpu.sync_copy(cval[p].at[pl.ds(j * SCW, wdt)],
                                acc_c.at[dix[p].at[pl.ds(j * SCW, wdt)]],
                                add=True)

        stage_d = [None, None]
        if full > 0:
            stage_d[0] = fire_stage(0, 0, RB)
        if full > 1:
            stage_d[1] = fire_stage(1, 1, RB)

        for k in range(full):
            p = k % 2
            stage_d[p][0].wait()
            stage_d[p][1].wait()
            stage_d[p] = None
            build_rows(p, RB)
            if k + 2 < full:
                stage_d[p] = fire_stage(k + 2, p, RB)
            elif tail and k + 2 == full:
                stage_d[p] = fire_stage(full, p, tail)
            fire_scatter(p, RB)

        if tail:
            p = full % 2
            if stage_d[p] is None:
                stage_d[p] = fire_stage(full, p, tail)
            stage_d[p][0].wait()
            stage_d[p][1].wait()
            build_rows(p, tail)
            fire_scatter(p, tail)

        if rem:
            @pl.when(has_extra)
            def _extra():
                r = row0 + i32(rpw)
                pltpu.sync_copy(src_hbm.at[pl.ds(r * i32(128), 128)],
                                six[0].at[pl.ds(0, 128)])
                pltpu.sync_copy(dst_hbm.at[pl.ds(r * i32(128), 128)],
                                dix[0].at[pl.ds(0, 128)])
                for g in range(8):
                    build_group(0, i32(0), g)
                fire_scatter(0, 1)

        plsc.subcore_barrier()
        pltpu.sync_copy(acc_a.at[pl.ds(s * i32(sl), sl)],
                        aout.at[c, pl.ds(s * i32(sl), sl)])
        pltpu.sync_copy(acc_c.at[pl.ds(s * i32(sl), sl)],
                        cout.at[c, pl.ds(s * i32(sl), sl)])

    return sc_kernel


def kernel(belief, probability, payoff_sample, edge_index):
    n = belief.shape[0]
    e = edge_index.shape[1]
    assert e % 128 == 0, "edge count must be a multiple of 128"
    n_pad = max(NS * 128, 1 << (n - 1).bit_length())
    rows2d = n_pad // 128

    b32 = belief.astype(jnp.float32)
    q32 = probability.astype(jnp.float32)
    p32 = payoff_sample.astype(jnp.int32)

    n4 = n_pad // 4
    b_flat = jnp.pad(b32, (0, n_pad - n))
    q_pad = jnp.pad(q32, (0, n_pad - n), constant_values=0.5).reshape(rows2d, 128)
    p_flat = jnp.pad(p32, (0, n_pad - n))
    b_pad = b_flat.reshape(rows2d, 128)
    e32 = edge_index.astype(jnp.int32)
    src32 = e32[0]
    dst32 = e32[1]
    zer = jnp.zeros((n_pad,), jnp.int32)

    tbl32 = pl.pallas_call(
        _prep_body,
        out_shape=jax.ShapeDtypeStruct((n4,), jnp.int32),
    )(*[b_flat[i * n4:(i + 1) * n4] for i in range(4)],
      *[p_flat[i * n4:(i + 1) * n4] for i in range(4)])

    acc_a, acc_c = _make_sc_kernel(n, n_pad, e)(tbl32, src32, dst32, zer)

    out = pl.pallas_call(
        _apply_body,
        out_shape=jax.ShapeDtypeStruct((rows2d, 128), jnp.float32),
    )(b_pad, q_pad,
      acc_a[0].reshape(rows2d, 128), acc_a[1].reshape(rows2d, 128),
      acc_c[0].reshape(rows2d, 128), acc_c[1].reshape(rows2d, 128))

    return out.reshape(n_pad)[:n].astype(jnp.float64)


# speedup vs baseline: 19.1283x; 1.2222x over previous
import functools

import jax
import jax.numpy as jnp
from jax import lax
from jax.experimental import pallas as pl
from jax.experimental.pallas import tpu as pltpu
from jax.experimental.pallas import tpu_sc as plsc

L = 16
NS = 16
NC = 2
NW = NC * NS
RB = 32
CHUNK = RB * 128
SCW = 512


def _prep_body(b0, b1, b2, b3, p0, p1, p2, p3, v_ref):
    def byte(b_ref, p_ref):
        mask = b_ref[...] > 0.5
        return jnp.where(mask, (p_ref[...] << 1) | 1, 0)
    v_ref[...] = (byte(b0, p0) | (byte(b1, p1) << 8)
                  | (byte(b2, p2) << 16) | (byte(b3, p3) << 24))


def _split_body(e_ref, s_ref, d_ref):
    x = e_ref[...]
    s_ref[...] = x[0]
    d_ref[...] = x[1]


def _apply_body(b_ref, q_ref, a0_ref, a1_ref, c0_ref, c1_ref, o_ref):
    b = b_ref[...]
    q = q_ref[...]
    a = (a0_ref[...] + a1_ref[...]).astype(jnp.float32)
    cnt = c0_ref[...] + c1_ref[...]
    t = jnp.exp(a * jnp.log((1.0 - q) / q))
    den = b + (1.0 - b) * t
    post = jnp.where(den > 0.0, b / den, b)
    o_ref[...] = jnp.where(cnt > 0, post, b)


def _make_sc_kernel(n, n_pad, e):
    i32 = jnp.int32
    N4LOG = (n_pad // 4).bit_length() - 1
    sl = n_pad // NS
    rows_total = e // 128
    rpw = rows_total // NW
    rem = rows_total % NW
    full = rpw // RB
    tail = rpw % RB
    mesh = plsc.VectorSubcoreMesh(
        core_axis_name="c", subcore_axis_name="s",
        num_cores=NC, num_subcores=NS)

    @functools.partial(
        pl.kernel,
        out_type=(jax.ShapeDtypeStruct((NC, n_pad), jnp.int32),
                  jax.ShapeDtypeStruct((NC, n_pad), jnp.int32)),
        mesh=mesh,
        compiler_params=pltpu.CompilerParams(needs_layout_passes=False),
        scratch_types=[
            pltpu.VMEM((n_pad // 4,), jnp.int32),
            [pltpu.VMEM((CHUNK,), jnp.int32) for _ in range(2)],
            [pltpu.VMEM((CHUNK,), jnp.int32) for _ in range(2)],
            [pltpu.VMEM((CHUNK,), jnp.int32) for _ in range(2)],
            [pltpu.VMEM((CHUNK,), jnp.int32) for _ in range(2)],
            [pltpu.SemaphoreType.DMA for _ in range(2)],
            pltpu.VMEM_SHARED((n_pad,), jnp.int32),
            pltpu.VMEM_SHARED((n_pad,), jnp.int32),
        ],
    )
    def sc_kernel(vpk_hbm, src_hbm, dst_hbm, zer_hbm, aout, cout,
                  table, six, dix, aval, cval, sem_stage, acc_a, acc_c):
        c = lax.axis_index("c")
        s = lax.axis_index("s")
        w = c * i32(NS) + s
        row0 = w * i32(rpw) + jnp.minimum(w, i32(rem))
        has_extra = w < i32(rem)

        pltpu.sync_copy(zer_hbm.at[pl.ds(s * i32(sl), sl)],
                        acc_a.at[pl.ds(s * i32(sl), sl)])
        pltpu.sync_copy(zer_hbm.at[pl.ds(s * i32(sl), sl)],
                        acc_c.at[pl.ds(s * i32(sl), sl)])
        pltpu.sync_copy(vpk_hbm, table)
        plsc.subcore_barrier()

        def fire_stage(k, p, nrows):
            r = row0 + i32(k * RB)
            return (
                pltpu.async_copy(src_hbm.at[pl.ds(r * i32(128), nrows * 128)],
                                 six[p].at[pl.ds(0, nrows * 128)],
                                 sem_stage[p]),
                pltpu.async_copy(dst_hbm.at[pl.ds(r * i32(128), nrows * 128)],
                                 dix[p].at[pl.ds(0, nrows * 128)],
                                 sem_stage[p]),
            )

        def build_group(p, r, g):
            sidx = six[p][pl.ds(r * i32(128) + i32(g * 16), 16)]
            word = plsc.load_gather(table, [sidx & i32(n_pad // 4 - 1)])
            v8 = (word >> ((sidx >> i32(N4LOG)) << i32(3))) & i32(0xFF)
            cnt = v8 & i32(1)
            a = i32(2) * (v8 >> i32(1)) - i32(10) * cnt
            o = r * i32(128) + i32(g * 16)
            aval[p][pl.ds(o, 16)] = a
            cval[p][pl.ds(o, 16)] = cnt

        def build_rows(p, nrows):
            def row_body(r, carry):
                for g in range(8):
                    build_group(p, r, g)
                return carry
            lax.fori_loop(i32(0), i32(nrows), row_body, i32(0))

        def fire_scatter(p, nrows):
            nedges = nrows * 128
            nsc = -(-nedges // SCW)
            for j in range(nsc):
                wdt = min(SCW, nedges - j * SCW)
                pltpu.sync_copy(aval[p].at[pl.ds(j * SCW, wdt)],
                                acc_a.at[dix[p].at[pl.ds(j * SCW, wdt)]],
                                add=True)
                pltpu.sync_copy(cval[p].at[pl.ds(j * SCW, wdt)],
                                acc_c.at[dix[p].at[pl.ds(j * SCW, wdt)]],
                                add=True)

        stage_d = [None, None]
        if full > 0:
            stage_d[0] = fire_stage(0, 0, RB)
        if full > 1:
            stage_d[1] = fire_stage(1, 1, RB)

        for k in range(full):
            p = k % 2
            stage_d[p][0].wait()
            stage_d[p][1].wait()
            stage_d[p] = None
            build_rows(p, RB)
            if k + 2 < full:
                stage_d[p] = fire_stage(k + 2, p, RB)
            elif tail and k + 2 == full:
                stage_d[p] = fire_stage(full, p, tail)
            fire_scatter(p, RB)

        if tail:
            p = full % 2
            if stage_d[p] is None:
                stage_d[p] = fire_stage(full, p, tail)
            stage_d[p][0].wait()
            stage_d[p][1].wait()
            build_rows(p, tail)
            fire_scatter(p, tail)

        if rem:
            @pl.when(has_extra)
            def _extra():
                r = row0 + i32(rpw)
                pltpu.sync_copy(src_hbm.at[pl.ds(r * i32(128), 128)],
                                six[0].at[pl.ds(0, 128)])
                pltpu.sync_copy(dst_hbm.at[pl.ds(r * i32(128), 128)],
                                dix[0].at[pl.ds(0, 128)])
                for g in range(8):
                    build_group(0, i32(0), g)
                fire_scatter(0, 1)

        plsc.subcore_barrier()
        pltpu.sync_copy(acc_a.at[pl.ds(s * i32(sl), sl)],
                        aout.at[c, pl.ds(s * i32(sl), sl)])
        pltpu.sync_copy(acc_c.at[pl.ds(s * i32(sl), sl)],
                        cout.at[c, pl.ds(s * i32(sl), sl)])

    return sc_kernel


def kernel(belief, probability, payoff_sample, edge_index):
    n = belief.shape[0]
    e = edge_index.shape[1]
    assert e % 128 == 0, "edge count must be a multiple of 128"
    n_pad = max(NS * 128, 1 << (n - 1).bit_length())
    rows2d = n_pad // 128

    b32 = belief.astype(jnp.float32)
    q32 = probability.astype(jnp.float32)
    p32 = payoff_sample.astype(jnp.int32)

    n4 = n_pad // 4
    b_flat = jnp.pad(b32, (0, n_pad - n))
    q_pad = jnp.pad(q32, (0, n_pad - n), constant_values=0.5).reshape(rows2d, 128)
    p_flat = jnp.pad(p32, (0, n_pad - n))
    b_pad = b_flat.reshape(rows2d, 128)
    e32 = edge_index.astype(jnp.int32)
    blk = 131072
    src32, dst32 = pl.pallas_call(
        _split_body,
        grid=(-(-e // blk),),
        in_specs=[pl.BlockSpec((2, blk), lambda i: (i * 0, i))],
        out_specs=[pl.BlockSpec((blk,), lambda i: (i,)),
                   pl.BlockSpec((blk,), lambda i: (i,))],
        out_shape=(jax.ShapeDtypeStruct((e,), jnp.int32),
                   jax.ShapeDtypeStruct((e,), jnp.int32)),
    )(e32)
    zer = jnp.zeros((n_pad,), jnp.int32)

    tbl32 = pl.pallas_call(
        _prep_body,
        out_shape=jax.ShapeDtypeStruct((n4,), jnp.int32),
    )(*[b_flat[i * n4:(i + 1) * n4] for i in range(4)],
      *[p_flat[i * n4:(i + 1) * n4] for i in range(4)])

    acc_a, acc_c = _make_sc_kernel(n, n_pad, e)(tbl32, src32, dst32, zer)

    out = pl.pallas_call(
        _apply_body,
        out_shape=jax.ShapeDtypeStruct((rows2d, 128), jnp.float32),
    )(b_pad, q_pad,
      acc_a[0].reshape(rows2d, 128), acc_a[1].reshape(rows2d, 128),
      acc_c[0].reshape(rows2d, 128), acc_c[1].reshape(rows2d, 128))

    return out.reshape(n_pad)[:n].astype(jnp.float64)


# speedup vs baseline: 19.1284x; 1.0000x over previous
import functools

import jax
import jax.numpy as jnp
from jax import lax
from jax.experimental import pallas as pl
from jax.experimental.pallas import tpu as pltpu
from jax.experimental.pallas import tpu_sc as plsc

L = 16
NS = 16
NC = 2
NW = NC * NS
RB = 32
CHUNK = RB * 128
SCW = 512


def _prep_body(b0, b1, b2, b3, p0, p1, p2, p3, v_ref):
    def byte(b_ref, p_ref):
        mask = b_ref[...] > 0.5
        return jnp.where(mask, (p_ref[...] << 1) | 1, 0)
    v_ref[...] = (byte(b0, p0) | (byte(b1, p1) << 8)
                  | (byte(b2, p2) << 16) | (byte(b3, p3) << 24))


def _split_body(e_ref, s_ref, d_ref):
    x = e_ref[...]
    s_ref[...] = x[0]
    d_ref[...] = x[1]


def _apply_body(b_ref, q_ref, a0_ref, a1_ref, c0_ref, c1_ref, o_ref):
    b = b_ref[...]
    q = q_ref[...]
    a = (a0_ref[...] + a1_ref[...]).astype(jnp.float32)
    cnt = c0_ref[...] + c1_ref[...]
    t = jnp.exp(a * jnp.log((1.0 - q) / q))
    den = b + (1.0 - b) * t
    post = jnp.where(den > 0.0, b / den, b)
    o_ref[...] = jnp.where(cnt > 0, post, b)


def _make_sc_kernel(n, n_pad, e):
    i32 = jnp.int32
    N4LOG = (n_pad // 4).bit_length() - 1
    sl = n_pad // NS
    rows_total = e // 128
    rpw = rows_total // NW
    rem = rows_total % NW
    full = rpw // RB
    tail = rpw % RB
    mesh = plsc.VectorSubcoreMesh(
        core_axis_name="c", subcore_axis_name="s",
        num_cores=NC, num_subcores=NS)

    @functools.partial(
        pl.kernel,
        out_type=(jax.ShapeDtypeStruct((NC, n_pad), jnp.int32),
                  jax.ShapeDtypeStruct((NC, n_pad), jnp.int32)),
        mesh=mesh,
        compiler_params=pltpu.CompilerParams(needs_layout_passes=False),
        scratch_types=[
            pltpu.VMEM((n_pad // 4,), jnp.int32),
            [pltpu.VMEM((CHUNK,), jnp.int32) for _ in range(2)],
            [pltpu.VMEM((CHUNK,), jnp.int32) for _ in range(2)],
            [pltpu.VMEM((CHUNK,), jnp.int32) for _ in range(2)],
            [pltpu.VMEM((CHUNK,), jnp.int32) for _ in range(2)],
            [pltpu.SemaphoreType.DMA for _ in range(2)],
            pltpu.VMEM_SHARED((n_pad,), jnp.int32),
            pltpu.VMEM_SHARED((n_pad,), jnp.int32),
        ],
    )
    def sc_kernel(vpk_hbm, src_hbm, dst_hbm, zer_hbm, aout, cout,
                  table, six, dix, aval, cval, sem_stage, acc_a, acc_c):
        c = lax.axis_index("c")
        s = lax.axis_index("s")
        w = c * i32(NS) + s
        row0 = w * i32(rpw) + jnp.minimum(w, i32(rem))
        has_extra = w < i32(rem)

        pltpu.sync_copy(zer_hbm.at[pl.ds(s * i32(sl), sl)],
                        acc_a.at[pl.ds(s * i32(sl), sl)])
        pltpu.sync_copy(zer_hbm.at[pl.ds(s * i32(sl), sl)],
                        acc_c.at[pl.ds(s * i32(sl), sl)])
        pltpu.sync_copy(vpk_hbm, table)
        plsc.subcore_barrier()

        def fire_stage(k, p, nrows):
            r = row0 + i32(k * RB)
            return (
                pltpu.async_copy(src_hbm.at[pl.ds(r * i32(128), nrows * 128)],
                                 six[p].at[pl.ds(0, nrows * 128)],
                                 sem_stage[p]),
                pltpu.async_copy(dst_hbm.at[pl.ds(r * i32(128), nrows * 128)],
                                 dix[p].at[pl.ds(0, nrows * 128)],
                                 sem_stage[p]),
            )

        def build_group(p, r, g):
            sidx = six[p][pl.ds(r * i32(128) + i32(g * 16), 16)]
            word = plsc.load_gather(table, [sidx & i32(n_pad // 4 - 1)])
            v8 = (word >> ((sidx >> i32(N4LOG)) << i32(3))) & i32(0xFF)
            cnt = v8 & i32(1)
            a = i32(2) * (v8 >> i32(1)) - i32(10) * cnt
            o = r * i32(128) + i32(g * 16)
            aval[p][pl.ds(o, 16)] = a
            cval[p][pl.ds(o, 16)] = cnt

        def build_rows(p, nrows):
            def row_body(r, carry):
                for g in range(8):
                    build_group(p, r, g)
                return carry
            lax.fori_loop(i32(0), i32(nrows), row_body, i32(0))

        def fire_scatter(p, nrows):
            nedges = nrows * 128
            nsc = -(-nedges // SCW)
            for j in range(nsc):
                wdt = min(SCW, nedges - j * SCW)
                pltpu.sync_copy(aval[p].at[pl.ds(j * SCW, wdt)],
                                acc_a.at[dix[p].at[pl.ds(j * SCW, wdt)]],
                                add=True)
                pltpu.sync_copy(cval[p].at[pl.ds(j * SCW, wdt)],
                                acc_c.at[dix[p].at[pl.ds(j * SCW, wdt)]],
                                add=True)

        stage_d = [None, None]
        if full > 0:
            stage_d[0] = fire_stage(0, 0, RB)
        if full > 1:
            stage_d[1] = fire_stage(1, 1, RB)

        for k in range(full):
            p = k % 2
            stage_d[p][0].wait()
            stage_d[p][1].wait()
            stage_d[p] = None
            build_rows(p, RB)
            fire_scatter(p, RB)
            if k + 2 < full:
                stage_d[p] = fire_stage(k + 2, p, RB)
            elif tail and k + 2 == full:
                stage_d[p] = fire_stage(full, p, tail)

        if tail:
            p = full % 2
            if stage_d[p] is None:
                stage_d[p] = fire_stage(full, p, tail)
            stage_d[p][0].wait()
            stage_d[p][1].wait()
            build_rows(p, tail)
            fire_scatter(p, tail)

        if rem:
            @pl.when(has_extra)
            def _extra():
                r = row0 + i32(rpw)
                pltpu.sync_copy(src_hbm.at[pl.ds(r * i32(128), 128)],
                                six[0].at[pl.ds(0, 128)])
                pltpu.sync_copy(dst_hbm.at[pl.ds(r * i32(128), 128)],
                                dix[0].at[pl.ds(0, 128)])
                for g in range(8):
                    build_group(0, i32(0), g)
                fire_scatter(0, 1)

        plsc.subcore_barrier()
        pltpu.sync_copy(acc_a.at[pl.ds(s * i32(sl), sl)],
                        aout.at[c, pl.ds(s * i32(sl), sl)])
        pltpu.sync_copy(acc_c.at[pl.ds(s * i32(sl), sl)],
                        cout.at[c, pl.ds(s * i32(sl), sl)])

    return sc_kernel


def kernel(belief, probability, payoff_sample, edge_index):
    n = belief.shape[0]
    e = edge_index.shape[1]
    assert e % 128 == 0, "edge count must be a multiple of 128"
    n_pad = max(NS * 128, 1 << (n - 1).bit_length())
    rows2d = n_pad // 128

    b32 = belief.astype(jnp.float32)
    q32 = probability.astype(jnp.float32)
    p32 = payoff_sample.astype(jnp.int32)

    n4 = n_pad // 4
    b_flat = jnp.pad(b32, (0, n_pad - n))
    q_pad = jnp.pad(q32, (0, n_pad - n), constant_values=0.5).reshape(rows2d, 128)
    p_flat = jnp.pad(p32, (0, n_pad - n))
    b_pad = b_flat.reshape(rows2d, 128)
    e32 = edge_index.astype(jnp.int32)
    blk = 131072
    src32, dst32 = pl.pallas_call(
        _split_body,
        grid=(-(-e // blk),),
        in_specs=[pl.BlockSpec((2, blk), lambda i: (i * 0, i))],
        out_specs=[pl.BlockSpec((blk,), lambda i: (i,)),
                   pl.BlockSpec((blk,), lambda i: (i,))],
        out_shape=(jax.ShapeDtypeStruct((e,), jnp.int32),
                   jax.ShapeDtypeStruct((e,), jnp.int32)),
    )(e32)
    zer = jnp.zeros((n_pad,), jnp.int32)

    tbl32 = pl.pallas_call(
        _prep_body,
        out_shape=jax.ShapeDtypeStruct((n4,), jnp.int32),
    )(*[b_flat[i * n4:(i + 1) * n4] for i in range(4)],
      *[p_flat[i * n4:(i + 1) * n4] for i in range(4)])

    acc_a, acc_c = _make_sc_kernel(n, n_pad, e)(tbl32, src32, dst32, zer)

    out = pl.pallas_call(
        _apply_body,
        out_shape=jax.ShapeDtypeStruct((rows2d, 128), jnp.float32),
    )(b_pad, q_pad,
      acc_a[0].reshape(rows2d, 128), acc_a[1].reshape(rows2d, 128),
      acc_c[0].reshape(rows2d, 128), acc_c[1].reshape(rows2d, 128))

    return out.reshape(n_pad)[:n].astype(jnp.float64)


# speedup vs baseline: 20.0780x; 1.0496x over previous
import functools

import jax
import jax.numpy as jnp
from jax import lax
from jax.experimental import pallas as pl
from jax.experimental.pallas import tpu as pltpu
from jax.experimental.pallas import tpu_sc as plsc

L = 16
NS = 16
NC = 2
NW = NC * NS
RB = 32
CHUNK = RB * 128
SCW = 512


def _prep_body(b0, b1, b2, b3, p0, p1, p2, p3, v_ref):
    def byte(b_ref, p_ref):
        mask = b_ref[...] > 0.5
        return jnp.where(mask, (p_ref[...] << 1) | 1, 0)
    v_ref[...] = (byte(b0, p0) | (byte(b1, p1) << 8)
                  | (byte(b2, p2) << 16) | (byte(b3, p3) << 24))


def _split_body(e_ref, s_ref, d_ref):
    x = e_ref[...]
    s_ref[...] = x[0]
    d_ref[...] = x[1]


def _apply_body(b_ref, q_ref, a0_ref, a1_ref, c0_ref, c1_ref, o_ref):
    b = b_ref[...]
    q = q_ref[...]
    a = (a0_ref[...] + a1_ref[...]).astype(jnp.float32)
    cnt = c0_ref[...] + c1_ref[...]
    t = jnp.exp(a * jnp.log((1.0 - q) / q))
    den = b + (1.0 - b) * t
    post = jnp.where(den > 0.0, b / den, b)
    o_ref[...] = jnp.where(cnt > 0, post, b)


def _make_sc_kernel(n, n_pad, e):
    i32 = jnp.int32
    N4LOG = (n_pad // 4).bit_length() - 1
    sl = n_pad // NS
    rows_total = e // 128
    rpw = rows_total // NW
    rem = rows_total % NW
    full = rpw // RB
    tail = rpw % RB
    mesh = plsc.VectorSubcoreMesh(
        core_axis_name="c", subcore_axis_name="s",
        num_cores=NC, num_subcores=NS)

    @functools.partial(
        pl.kernel,
        out_type=(jax.ShapeDtypeStruct((NC, n_pad), jnp.int32),
                  jax.ShapeDtypeStruct((NC, n_pad), jnp.int32)),
        mesh=mesh,
        compiler_params=pltpu.CompilerParams(needs_layout_passes=False),
        scratch_types=[
            pltpu.VMEM((n_pad // 4,), jnp.int32),
            [pltpu.VMEM((CHUNK,), jnp.int32) for _ in range(2)],
            [pltpu.VMEM((CHUNK,), jnp.int32) for _ in range(2)],
            [pltpu.VMEM((CHUNK,), jnp.int32) for _ in range(2)],
            [pltpu.VMEM((CHUNK,), jnp.int32) for _ in range(2)],
            [pltpu.SemaphoreType.DMA for _ in range(2)],
            [pltpu.SemaphoreType.DMA for _ in range(2)],
            pltpu.VMEM_SHARED((n_pad,), jnp.int32),
            pltpu.VMEM_SHARED((n_pad,), jnp.int32),
        ],
    )
    def sc_kernel(vpk_hbm, src_hbm, dst_hbm, zer_hbm, aout, cout,
                  table, six, dix, aval, cval, sem_stage, sem_scat,
                  acc_a, acc_c):
        c = lax.axis_index("c")
        s = lax.axis_index("s")
        w = c * i32(NS) + s
        row0 = w * i32(rpw) + jnp.minimum(w, i32(rem))
        has_extra = w < i32(rem)

        pltpu.sync_copy(zer_hbm.at[pl.ds(s * i32(sl), sl)],
                        acc_a.at[pl.ds(s * i32(sl), sl)])
        pltpu.sync_copy(zer_hbm.at[pl.ds(s * i32(sl), sl)],
                        acc_c.at[pl.ds(s * i32(sl), sl)])
        pltpu.sync_copy(vpk_hbm, table)
        plsc.subcore_barrier()

        def fire_stage(k, p, nrows):
            r = row0 + i32(k * RB)
            return (
                pltpu.async_copy(src_hbm.at[pl.ds(r * i32(128), nrows * 128)],
                                 six[p].at[pl.ds(0, nrows * 128)],
                                 sem_stage[p]),
                pltpu.async_copy(dst_hbm.at[pl.ds(r * i32(128), nrows * 128)],
                                 dix[p].at[pl.ds(0, nrows * 128)],
                                 sem_stage[p]),
            )

        def build_group(p, r, g):
            sidx = six[p][pl.ds(r * i32(128) + i32(g * 16), 16)]
            word = plsc.load_gather(table, [sidx & i32(n_pad // 4 - 1)])
            v8 = (word >> ((sidx >> i32(N4LOG)) << i32(3))) & i32(0xFF)
            cnt = v8 & i32(1)
            a = i32(2) * (v8 >> i32(1)) - i32(10) * cnt
            o = r * i32(128) + i32(g * 16)
            aval[p][pl.ds(o, 16)] = a
            cval[p][pl.ds(o, 16)] = cnt

        def build_rows(p, nrows):
            def row_body(r, carry):
                for g in range(8):
                    build_group(p, r, g)
                return carry
            lax.fori_loop(i32(0), i32(nrows), row_body, i32(0))

        def fire_scatter(p, nrows):
            nedges = nrows * 128
            nsc = -(-nedges // SCW)
            descs = []
            for j in range(nsc):
                wdt = min(SCW, nedges - j * SCW)
                descs.append(pltpu.async_copy(
                    aval[p].at[pl.ds(j * SCW, wdt)],
                    acc_a.at[dix[p].at[pl.ds(j * SCW, wdt)]],
                    sem_scat[p], add=True))
                descs.append(pltpu.async_copy(
                    cval[p].at[pl.ds(j * SCW, wdt)],
                    acc_c.at[dix[p].at[pl.ds(j * SCW, wdt)]],
                    sem_scat[p], add=True))
            for d in descs:
                d.wait()

        stage_d = [None, None]
        if full > 0:
            stage_d[0] = fire_stage(0, 0, RB)
        if full > 1:
            stage_d[1] = fire_stage(1, 1, RB)

        for k in range(full):
            p = k % 2
            stage_d[p][0].wait()
            stage_d[p][1].wait()
            stage_d[p] = None
            build_rows(p, RB)
            fire_scatter(p, RB)
            if k + 2 < full:
                stage_d[p] = fire_stage(k + 2, p, RB)
            elif tail and k + 2 == full:
                stage_d[p] = fire_stage(full, p, tail)

        if tail:
            p = full % 2
            if stage_d[p] is None:
                stage_d[p] = fire_stage(full, p, tail)
            stage_d[p][0].wait()
            stage_d[p][1].wait()
            build_rows(p, tail)
            fire_scatter(p, tail)

        if rem:
            @pl.when(has_extra)
            def _extra():
                r = row0 + i32(rpw)
                pltpu.sync_copy(src_hbm.at[pl.ds(r * i32(128), 128)],
                                six[0].at[pl.ds(0, 128)])
                pltpu.sync_copy(dst_hbm.at[pl.ds(r * i32(128), 128)],
                                dix[0].at[pl.ds(0, 128)])
                for g in range(8):
                    build_group(0, i32(0), g)
                fire_scatter(0, 1)

        plsc.subcore_barrier()
        pltpu.sync_copy(acc_a.at[pl.ds(s * i32(sl), sl)],
                        aout.at[c, pl.ds(s * i32(sl), sl)])
        pltpu.sync_copy(acc_c.at[pl.ds(s * i32(sl), sl)],
                        cout.at[c, pl.ds(s * i32(sl), sl)])

    return sc_kernel


def kernel(belief, probability, payoff_sample, edge_index):
    n = belief.shape[0]
    e = edge_index.shape[1]
    assert e % 128 == 0, "edge count must be a multiple of 128"
    n_pad = max(NS * 128, 1 << (n - 1).bit_length())
    rows2d = n_pad // 128

    b32 = belief.astype(jnp.float32)
    q32 = probability.astype(jnp.float32)
    p32 = payoff_sample.astype(jnp.int32)

    n4 = n_pad // 4
    b_flat = jnp.pad(b32, (0, n_pad - n))
    q_pad = jnp.pad(q32, (0, n_pad - n), constant_values=0.5).reshape(rows2d, 128)
    p_flat = jnp.pad(p32, (0, n_pad - n))
    b_pad = b_flat.reshape(rows2d, 128)
    e32 = edge_index.astype(jnp.int32)
    blk = 131072
    src32, dst32 = pl.pallas_call(
        _split_body,
        grid=(-(-e // blk),),
        in_specs=[pl.BlockSpec((2, blk), lambda i: (i * 0, i))],
        out_specs=[pl.BlockSpec((blk,), lambda i: (i,)),
                   pl.BlockSpec((blk,), lambda i: (i,))],
        out_shape=(jax.ShapeDtypeStruct((e,), jnp.int32),
                   jax.ShapeDtypeStruct((e,), jnp.int32)),
    )(e32)
    zer = jnp.zeros((n_pad,), jnp.int32)

    tbl32 = pl.pallas_call(
        _prep_body,
        out_shape=jax.ShapeDtypeStruct((n4,), jnp.int32),
    )(*[b_flat[i * n4:(i + 1) * n4] for i in range(4)],
      *[p_flat[i * n4:(i + 1) * n4] for i in range(4)])

    acc_a, acc_c = _make_sc_kernel(n, n_pad, e)(tbl32, src32, dst32, zer)

    out = pl.pallas_call(
        _apply_body,
        out_shape=jax.ShapeDtypeStruct((rows2d, 128), jnp.float32),
    )(b_pad, q_pad,
      acc_a[0].reshape(rows2d, 128), acc_a[1].reshape(rows2d, 128),
      acc_c[0].reshape(rows2d, 128), acc_c[1].reshape(rows2d, 128))

    return out.reshape(n_pad)[:n].astype(jnp.float64)
